# baseline (speedup 1.0000x reference)
import functools

import jax
import jax.numpy as jnp
from jax import lax
from jax.experimental import pallas as pl
from jax.experimental.pallas import tpu as pltpu
from jax.experimental.pallas import tpu_sc as plsc

NC = 2
NS = 16
NW = NC * NS
LANES = 16
CHUNK = 128
NBUF = 2
NGRAPHS = 128


def _sc_mesh():
    return plsc.VectorSubcoreMesh(core_axis_name="c", subcore_axis_name="s")


def _fill_rows(ref, nrows, width, value):
    @pl.loop(0, nrows)
    def _(i):
        @pl.loop(0, width, step=LANES)
        def _(j):
            ref[i, pl.ds(j, LANES)] = jnp.full((LANES,), value, jnp.float32)


def _zero_acc(zero_v, acc_sh, row0, stripe, zrows):
    nfull = stripe // zrows
    rem = stripe - nfull * zrows

    @pl.loop(0, nfull)
    def _(j):
        pltpu.sync_copy(zero_v, acc_sh.at[pl.ds(row0 + j * zrows, zrows)])

    if rem:
        pltpu.sync_copy(zero_v.at[pl.ds(0, rem)],
                        acc_sh.at[pl.ds(row0 + nfull * zrows, rem)])


def _deg_call(nr, k_ch, nreal):
    sw = nr // NS
    g16 = CHUNK // LANES

    @functools.partial(
        pl.kernel,
        mesh=_sc_mesh(),
        out_type=jax.ShapeDtypeStruct((NC, nr), jnp.float32),
        compiler_params=pltpu.CompilerParams(needs_layout_passes=False),
        scratch_types=[
            pltpu.VMEM((nr,), jnp.float32),
            pltpu.VMEM((CHUNK,), jnp.int32),
            pltpu.VMEM((CHUNK,), jnp.int32),
            pltpu.VMEM((NS, sw), jnp.float32),
            pltpu.VMEM_SHARED((NS, nr), jnp.float32),
            pltpu.SemaphoreType.DMA,
            pltpu.SemaphoreType.DMA,
        ],
    )
    def k(edge_hbm, pad_hbm, out_hbm, hist, didx0, didx1, red_v, acc_sh,
          is0, is1):
        didx = [didx0, didx1]
        isem = [is0, is1]
        c = lax.axis_index("c")
        s = lax.axis_index("s")
        wid = c * NS + s

        def issue(gk, b):
            @pl.when(gk < nreal)
            def _():
                pltpu.async_copy(
                    edge_hbm.at[1, pl.ds(gk * CHUNK, CHUNK)], didx[b],
                    isem[b])

            @pl.when(gk >= nreal)
            def _():
                pltpu.async_copy(
                    pad_hbm.at[pl.ds((gk - nreal) * CHUNK, CHUNK)], didx[b],
                    isem[b])

        @pl.loop(0, nr // LANES)
        def _(i):
            hist[pl.ds(i * LANES, LANES)] = jnp.zeros((LANES,), jnp.float32)

        ones16 = jnp.ones((LANES,), jnp.float32)
        for b in range(NBUF):
            issue(wid * k_ch + b, b)

        @pl.loop(0, (k_ch - NBUF) // NBUF)
        def _(j):
            for b in range(NBUF):
                kk = j * NBUF + b
                pltpu.make_async_copy(
                    pad_hbm.at[pl.ds(0, CHUNK)], didx[b], isem[b]).wait()
                for o in range(g16):
                    idx = didx[b][pl.ds(o * LANES, LANES)]
                    plsc.addupdate_scatter(hist, [idx], ones16)
                issue(wid * k_ch + kk + NBUF, b)

        for b in range(NBUF):
            pltpu.make_async_copy(
                pad_hbm.at[pl.ds(0, CHUNK)], didx[b], isem[b]).wait()
            for o in range(g16):
                idx = didx[b][pl.ds(o * LANES, LANES)]
                plsc.addupdate_scatter(hist, [idx], ones16)

        pltpu.sync_copy(hist, acc_sh.at[s])
        plsc.subcore_barrier()
        base = s * sw
        for w in range(NS):
            pltpu.async_copy(acc_sh.at[w, pl.ds(base, sw)], red_v.at[w], is0)
        for w in range(NS):
            pltpu.make_async_copy(
                acc_sh.at[w, pl.ds(base, sw)], red_v.at[w], is0).wait()

        @pl.loop(0, sw // LANES)
        def _(j):
            t = red_v[0, pl.ds(j * LANES, LANES)]
            for w in range(1, NS):
                t = t + red_v[w, pl.ds(j * LANES, LANES)]
            hist[pl.ds(j * LANES, LANES)] = t

        pltpu.sync_copy(hist.at[pl.ds(0, sw)], out_hbm.at[c, pl.ds(base, sw)])

    return k


def _agg_call(nr, k_ch, h):
    stripe = nr // NS
    zrows = 16

    @functools.partial(
        pl.kernel,
        mesh=_sc_mesh(),
        out_type=jax.ShapeDtypeStruct((NC, nr, h), jnp.float32),
        scratch_types=[
            pltpu.VMEM((k_ch, CHUNK), jnp.int32),
            pltpu.VMEM((CHUNK,), jnp.int32),
            pltpu.VMEM((CHUNK,), jnp.int32),
            pltpu.VMEM((CHUNK, h), jnp.float32),
            pltpu.VMEM((CHUNK, h), jnp.float32),
            pltpu.VMEM((zrows, h), jnp.float32),
            pltpu.VMEM_SHARED((nr, h), jnp.float32),
            pltpu.SemaphoreType.DMA,
            pltpu.SemaphoreType.DMA,
            pltpu.SemaphoreType.DMA,
            pltpu.SemaphoreType.DMA,
        ],
    )
    def k(y_hbm, src_hbm, dst_hbm, out_hbm, src_v, didx0, didx1, rows0, rows1,
          zero_v, acc_sh, gs0, gs1, is0, is1):
        didx = [didx0, didx1]
        rows = [rows0, rows1]
        gsem = [gs0, gs1]
        isem = [is0, is1]
        c = lax.axis_index("c")
        s = lax.axis_index("s")
        wid = c * NS + s
        _fill_rows(zero_v, zrows, h, 0.0)
        row0 = s * stripe
        _zero_acc(zero_v, acc_sh, row0, stripe, zrows)
        plsc.subcore_barrier()
        pltpu.sync_copy(src_hbm.at[wid], src_v)

        for b in range(NBUF):
            pltpu.async_copy(dst_hbm.at[wid, b], didx[b], isem[b])
            pltpu.async_copy(y_hbm.at[src_v.at[b]], rows[b], gsem[b])

        @pl.loop(0, (k_ch - NBUF) // NBUF)
        def _(j):
            for b in range(NBUF):
                kk = j * NBUF + b
                pltpu.make_async_copy(
                    dst_hbm.at[wid, 0], didx[b], isem[b]).wait()
                pltpu.make_async_copy(
                    y_hbm.at[src_v.at[0]], rows[b], gsem[b]).wait()
                pltpu.sync_copy(rows[b], acc_sh.at[didx[b]], add=True)
                pltpu.async_copy(dst_hbm.at[wid, kk + NBUF], didx[b], isem[b])
                pltpu.async_copy(
                    y_hbm.at[src_v.at[kk + NBUF]], rows[b], gsem[b])

        for b in range(NBUF):
            pltpu.make_async_copy(dst_hbm.at[wid, 0], didx[b], isem[b]).wait()
            pltpu.make_async_copy(
                y_hbm.at[src_v.at[0]], rows[b], gsem[b]).wait()
            pltpu.sync_copy(rows[b], acc_sh.at[didx[b]], add=True)

        plsc.subcore_barrier()
        pltpu.sync_copy(acc_sh.at[pl.ds(row0, stripe)],
                        out_hbm.at[c, pl.ds(row0, stripe)])

    return k


def _tc_matmul(xp, w):
    nr = xp.shape[0]
    h = w.shape[1]

    def body(x_ref, w_ref, o_ref):
        o_ref[...] = jnp.dot(x_ref[...], w_ref[...],
                             preferred_element_type=jnp.float32)

    return pl.pallas_call(
        body, out_shape=jax.ShapeDtypeStruct((nr, h), jnp.float32))(xp, w)


def _dinv_col(deg_ref, nr, h):
    dt = deg_ref[0:nr, :] + deg_ref[nr:2 * nr, :] + 1.0
    return jnp.broadcast_to(lax.rsqrt(dt), (nr, h))


def _tc_scale(deg2, xw):
    nr, h = xw.shape

    def body(deg_ref, xw_ref, y_ref):
        y_ref[...] = _dinv_col(deg_ref, nr, h) * xw_ref[...]

    return pl.pallas_call(
        body, out_shape=jax.ShapeDtypeStruct((nr, h), jnp.float32))(deg2, xw)


def _tc_layer_mid(agg_part, y1, deg2, w2, b1):
    nr, h = y1.shape

    def body(p_ref, y_ref, deg_ref, w_ref, b_ref, o_ref):
        dinv = _dinv_col(deg_ref, nr, h)
        agg = p_ref[0] + p_ref[1] + y_ref[...]
        h1 = jnp.maximum(dinv * agg + b_ref[...], 0.0)
        o_ref[...] = dinv * jnp.dot(
            h1, w_ref[...], preferred_element_type=jnp.float32)

    return pl.pallas_call(
        body, out_shape=jax.ShapeDtypeStruct((nr, h), jnp.float32))(
            agg_part, y1, deg2, w2, b1)


def _tc_last_pool(agg_part, y2, deg2, b2, starts, g):
    nc, nr, h = agg_part.shape
    gpb = 8

    def body(st_sref, p_ref, y_ref, deg_ref, b_ref, o_ref):
        i0 = pl.program_id(0) * gpb
        bias = b_ref[...]
        for r in range(gpb):
            s0 = st_sref[i0 + r]
            s1 = st_sref[i0 + r + 1]
            nblk = (s1 - s0 + 7) // 8

            def step(i, acc, s0=s0, s1=s1):
                base = s0 + i * 8
                dt = (deg_ref[pl.ds(base, 8), :]
                      + deg_ref[pl.ds(nr + base, 8), :] + 1.0)
                dinv = jnp.broadcast_to(lax.rsqrt(dt), (8, h))
                agg = (p_ref[0, pl.ds(base, 8), :]
                       + p_ref[1, pl.ds(base, 8), :]
                       + y_ref[pl.ds(base, 8), :])
                rows = jnp.maximum(dinv * agg + bias, 0.0)
                mask = (base + lax.broadcasted_iota(
                    jnp.int32, (8, 1), 0)) < s1
                return jnp.maximum(acc, jnp.where(mask, rows, -jnp.inf))

            acc = lax.fori_loop(0, nblk, step,
                                jnp.full((8, h), -jnp.inf, jnp.float32))
            o_ref[r, :] = jnp.max(acc, axis=0)

    grid_spec = pltpu.PrefetchScalarGridSpec(
        num_scalar_prefetch=1,
        grid=(g // gpb,),
        in_specs=[
            pl.BlockSpec((nc, nr, h), lambda i, st: (0, 0, 0)),
            pl.BlockSpec((nr, h), lambda i, st: (0, 0)),
            pl.BlockSpec((nc * nr, 1), lambda i, st: (0, 0)),
            pl.BlockSpec((1, h), lambda i, st: (0, 0)),
        ],
        out_specs=pl.BlockSpec((gpb, h), lambda i, st: (i, 0)),
    )
    return pl.pallas_call(
        body, grid_spec=grid_spec,
        out_shape=jax.ShapeDtypeStruct((g, h), jnp.float32))(
            starts, agg_part, y2, deg2, b2)


def _tc_pool(h2p, starts, g):
    nrp, h = h2p.shape
    gpb = 8

    def body(st_sref, h_ref, o_ref):
        i0 = pl.program_id(0) * gpb
        for r in range(gpb):
            s0 = st_sref[i0 + r]
            s1 = st_sref[i0 + r + 1]
            nblk = (s1 - s0 + 7) // 8

            def step(i, acc, s0=s0, s1=s1):
                base = s0 + i * 8
                rows = h_ref[pl.ds(base, 8), :]
                mask = (base + lax.broadcasted_iota(
                    jnp.int32, (8, 1), 0)) < s1
                return jnp.maximum(acc, jnp.where(mask, rows, -jnp.inf))

            acc = lax.fori_loop(0, nblk, step,
                                jnp.full((8, h), -jnp.inf, jnp.float32))
            o_ref[r, :] = jnp.max(acc, axis=0)

    grid_spec = pltpu.PrefetchScalarGridSpec(
        num_scalar_prefetch=1,
        grid=(g // gpb,),
        in_specs=[pl.BlockSpec((nrp, h), lambda i, st: (0, 0))],
        out_specs=pl.BlockSpec((gpb, h), lambda i, st: (i, 0)),
    )
    return pl.pallas_call(
        body, grid_spec=grid_spec,
        out_shape=jax.ShapeDtypeStruct((g, h), jnp.float32))(starts, h2p)


def _tc_final(pooled, wlin, blin):
    g = pooled.shape[0]
    cc = wlin.shape[1]

    def body(p_ref, w_ref, b_ref, o_ref):
        o_ref[...] = jnp.dot(p_ref[...], w_ref[...],
                             preferred_element_type=jnp.float32) + b_ref[...]

    return pl.pallas_call(
        body, out_shape=jax.ShapeDtypeStruct((g, cc), jnp.float32))(
            pooled, wlin, blin)


def kernel(x, edge_index, batch, W1, b1, W2, b2, Wlin, blin):
    n, d = x.shape
    h = W1.shape[1]
    e = edge_index.shape[1]
    g = NGRAPHS

    nra = NS * 128
    nr = ((n + 8 + nra - 1) // nra) * nra

    e_per_w = -(-e // NW)
    k_ch = -(-e_per_w // CHUNK)
    k_ch = max(2 * NBUF, -(-k_ch // NBUF) * NBUF)
    e_pad = NW * k_ch * CHUNK
    npad = e_pad - e
    pidx = jnp.arange(npad, dtype=jnp.int32)
    src = jnp.concatenate([edge_index[0], pidx % jnp.int32(n)])
    dst = jnp.concatenate([edge_index[1], jnp.int32(n) + pidx % jnp.int32(nr - n)])
    src3 = src.reshape(NW, k_ch, CHUNK)
    dst3 = dst.reshape(NW, k_ch, CHUNK)
    nreal = e // CHUNK
    pad_dst = jnp.concatenate([
        edge_index[1, nreal * CHUNK:],
        jnp.int32(n) + pidx % jnp.int32(nr - n)])
    xp = jnp.concatenate([x, jnp.zeros((nr - n, d), jnp.float32)])
    b1r = b1.reshape(1, h)
    b2r = b2.reshape(1, h)
    blinr = blin.reshape(1, -1)
    starts = jnp.searchsorted(
        batch, jnp.arange(g + 1, dtype=jnp.int32), side="left"
    ).astype(jnp.int32)

    deg = _deg_call(nr, k_ch, nreal)(edge_index, pad_dst)
    deg2 = deg.reshape(NC * nr, 1)
    xw1 = _tc_matmul(xp, W1)
    y1 = _tc_scale(deg2, xw1)
    agg1 = _agg_call(nr, k_ch, h)(y1, src3, dst3)
    y2 = _tc_layer_mid(agg1, y1, deg2, W2, b1r)
    agg2 = _agg_call(nr, k_ch, h)(y2, src3, dst3)
    pooled = _tc_last_pool(agg2, y2, deg2, b2r, starts, g)
    return _tc_final(pooled, Wlin, blinr)


# speedup vs baseline: 1.2944x; 1.2944x over previous
import functools

import jax
import jax.numpy as jnp
from jax import lax
from jax.experimental import pallas as pl
from jax.experimental.pallas import tpu as pltpu
from jax.experimental.pallas import tpu_sc as plsc

NC = 2
NS = 16
NW = NC * NS
LANES = 16
CHUNK = 128
NBUF = 2
NGRAPHS = 128


def _sc_mesh():
    return plsc.VectorSubcoreMesh(core_axis_name="c", subcore_axis_name="s")


def _fill_rows(ref, nrows, width, value):
    @pl.loop(0, nrows)
    def _(i):
        @pl.loop(0, width, step=LANES)
        def _(j):
            ref[i, pl.ds(j, LANES)] = jnp.full((LANES,), value, jnp.float32)


def _zero_acc(zero_v, acc_sh, row0, stripe, zrows):
    nfull = stripe // zrows
    rem = stripe - nfull * zrows

    @pl.loop(0, nfull)
    def _(j):
        pltpu.sync_copy(zero_v, acc_sh.at[pl.ds(row0 + j * zrows, zrows)])

    if rem:
        pltpu.sync_copy(zero_v.at[pl.ds(0, rem)],
                        acc_sh.at[pl.ds(row0 + nfull * zrows, rem)])


def _deg_call(nr, k_ch, nreal):
    sw = nr // NS
    g16 = CHUNK // LANES

    @functools.partial(
        pl.kernel,
        mesh=_sc_mesh(),
        out_type=jax.ShapeDtypeStruct((NC, nr), jnp.float32),
        compiler_params=pltpu.CompilerParams(needs_layout_passes=False),
        scratch_types=[
            pltpu.VMEM((nr,), jnp.float32),
            pltpu.VMEM((CHUNK,), jnp.int32),
            pltpu.VMEM((CHUNK,), jnp.int32),
            pltpu.VMEM((NS, sw), jnp.float32),
            pltpu.VMEM_SHARED((NS, nr), jnp.float32),
            pltpu.SemaphoreType.DMA,
            pltpu.SemaphoreType.DMA,
        ],
    )
    def k(edge_hbm, pad_hbm, out_hbm, hist, didx0, didx1, red_v, acc_sh,
          is0, is1):
        didx = [didx0, didx1]
        isem = [is0, is1]
        c = lax.axis_index("c")
        s = lax.axis_index("s")
        wid = c * NS + s

        def issue(gk, b):
            @pl.when(gk < nreal)
            def _():
                pltpu.async_copy(
                    edge_hbm.at[1, pl.ds(gk * CHUNK, CHUNK)], didx[b],
                    isem[b])

            @pl.when(gk >= nreal)
            def _():
                pltpu.async_copy(
                    pad_hbm.at[pl.ds((gk - nreal) * CHUNK, CHUNK)], didx[b],
                    isem[b])

        @pl.loop(0, nr // LANES)
        def _(i):
            hist[pl.ds(i * LANES, LANES)] = jnp.zeros((LANES,), jnp.float32)

        ones16 = jnp.ones((LANES,), jnp.float32)
        for b in range(NBUF):
            issue(wid * k_ch + b, b)

        @pl.loop(0, (k_ch - NBUF) // NBUF)
        def _(j):
            for b in range(NBUF):
                kk = j * NBUF + b
                pltpu.make_async_copy(
                    pad_hbm.at[pl.ds(0, CHUNK)], didx[b], isem[b]).wait()
                for o in range(g16):
                    idx = didx[b][pl.ds(o * LANES, LANES)]
                    plsc.addupdate_scatter(hist, [idx], ones16)
                issue(wid * k_ch + kk + NBUF, b)

        for b in range(NBUF):
            pltpu.make_async_copy(
                pad_hbm.at[pl.ds(0, CHUNK)], didx[b], isem[b]).wait()
            for o in range(g16):
                idx = didx[b][pl.ds(o * LANES, LANES)]
                plsc.addupdate_scatter(hist, [idx], ones16)

        pltpu.sync_copy(hist, acc_sh.at[s])
        plsc.subcore_barrier()
        base = s * sw
        for w in range(NS):
            pltpu.async_copy(acc_sh.at[w, pl.ds(base, sw)], red_v.at[w], is0)
        for w in range(NS):
            pltpu.make_async_copy(
                acc_sh.at[w, pl.ds(base, sw)], red_v.at[w], is0).wait()

        @pl.loop(0, sw // LANES)
        def _(j):
            t = red_v[0, pl.ds(j * LANES, LANES)]
            for w in range(1, NS):
                t = t + red_v[w, pl.ds(j * LANES, LANES)]
            hist[pl.ds(j * LANES, LANES)] = t

        pltpu.sync_copy(hist.at[pl.ds(0, sw)], out_hbm.at[c, pl.ds(base, sw)])

    return k


def _agg_call(nr, k_ch, h):
    stripe = nr // NS
    zrows = 16

    @functools.partial(
        pl.kernel,
        mesh=_sc_mesh(),
        out_type=jax.ShapeDtypeStruct((NC, nr, h), jnp.float32),
        scratch_types=[
            pltpu.VMEM((k_ch, CHUNK), jnp.int32),
            pltpu.VMEM((CHUNK,), jnp.int32),
            pltpu.VMEM((CHUNK,), jnp.int32),
            pltpu.VMEM((CHUNK, h), jnp.float32),
            pltpu.VMEM((CHUNK, h), jnp.float32),
            pltpu.VMEM((zrows, h), jnp.float32),
            pltpu.VMEM_SHARED((nr, h), jnp.float32),
            pltpu.SemaphoreType.DMA,
            pltpu.SemaphoreType.DMA,
            pltpu.SemaphoreType.DMA,
            pltpu.SemaphoreType.DMA,
        ],
    )
    def k(y_hbm, src_hbm, dst_hbm, out_hbm, src_v, didx0, didx1, rows0, rows1,
          zero_v, acc_sh, gs0, gs1, is0, is1):
        didx = [didx0, didx1]
        rows = [rows0, rows1]
        gsem = [gs0, gs1]
        isem = [is0, is1]
        c = lax.axis_index("c")
        s = lax.axis_index("s")
        wid = c * NS + s
        _fill_rows(zero_v, zrows, h, 0.0)
        row0 = s * stripe
        _zero_acc(zero_v, acc_sh, row0, stripe, zrows)
        plsc.subcore_barrier()
        pltpu.sync_copy(src_hbm.at[wid], src_v)

        for b in range(NBUF):
            pltpu.async_copy(dst_hbm.at[wid, b], didx[b], isem[b])
            pltpu.async_copy(y_hbm.at[src_v.at[b]], rows[b], gsem[b])

        @pl.loop(0, (k_ch - NBUF) // NBUF)
        def _(j):
            for b in range(NBUF):
                kk = j * NBUF + b
                pltpu.make_async_copy(
                    dst_hbm.at[wid, 0], didx[b], isem[b]).wait()
                pltpu.make_async_copy(
                    y_hbm.at[src_v.at[0]], rows[b], gsem[b]).wait()
                pltpu.sync_copy(rows[b], acc_sh.at[didx[b]], add=True)
                pltpu.async_copy(dst_hbm.at[wid, kk + NBUF], didx[b], isem[b])
                pltpu.async_copy(
                    y_hbm.at[src_v.at[kk + NBUF]], rows[b], gsem[b])

        for b in range(NBUF):
            pltpu.make_async_copy(dst_hbm.at[wid, 0], didx[b], isem[b]).wait()
            pltpu.make_async_copy(
                y_hbm.at[src_v.at[0]], rows[b], gsem[b]).wait()
            pltpu.sync_copy(rows[b], acc_sh.at[didx[b]], add=True)

        plsc.subcore_barrier()
        pltpu.sync_copy(acc_sh.at[pl.ds(row0, stripe)],
                        out_hbm.at[c, pl.ds(row0, stripe)])

    return k


def _tc_matmul(xp, w):
    nr = xp.shape[0]
    h = w.shape[1]

    def body(x_ref, w_ref, o_ref):
        o_ref[...] = jnp.dot(x_ref[...], w_ref[...],
                             preferred_element_type=jnp.float32)

    return pl.pallas_call(
        body, out_shape=jax.ShapeDtypeStruct((nr, h), jnp.float32))(xp, w)


def _dinv_col(deg_ref, nr, h):
    dt = deg_ref[0:nr, :] + deg_ref[nr:2 * nr, :] + 1.0
    return jnp.broadcast_to(lax.rsqrt(dt), (nr, h))


def _tc_scale(deg2, xw):
    nr, h = xw.shape

    def body(deg_ref, xw_ref, y_ref):
        y_ref[...] = _dinv_col(deg_ref, nr, h) * xw_ref[...]

    return pl.pallas_call(
        body, out_shape=jax.ShapeDtypeStruct((nr, h), jnp.float32))(deg2, xw)


def _tc_layer_mid(agg_part, y1, deg2, w2, b1):
    nr, h = y1.shape

    def body(p_ref, y_ref, deg_ref, w_ref, b_ref, o_ref):
        dinv = _dinv_col(deg_ref, nr, h)
        agg = p_ref[0] + p_ref[1] + y_ref[...]
        h1 = jnp.maximum(dinv * agg + b_ref[...], 0.0)
        o_ref[...] = dinv * jnp.dot(
            h1, w_ref[...], preferred_element_type=jnp.float32)

    return pl.pallas_call(
        body, out_shape=jax.ShapeDtypeStruct((nr, h), jnp.float32))(
            agg_part, y1, deg2, w2, b1)


def _tc_layer_last(agg_part, y2, deg2, b2):
    nr, h = y2.shape

    def body(p_ref, y_ref, deg_ref, b_ref, o_ref):
        dinv = _dinv_col(deg_ref, nr, h)
        agg = p_ref[0] + p_ref[1] + y_ref[...]
        o_ref[...] = jnp.maximum(dinv * agg + b_ref[...], 0.0)

    return pl.pallas_call(
        body, out_shape=jax.ShapeDtypeStruct((nr, h), jnp.float32))(
            agg_part, y2, deg2, b2)


def _tc_pool(h2p, starts, g):
    nrp, h = h2p.shape
    gpb = 8

    def body(st_sref, h_ref, o_ref):
        i0 = pl.program_id(0) * gpb
        for r in range(gpb):
            s0 = st_sref[i0 + r]
            s1 = st_sref[i0 + r + 1]
            nblk = (s1 - s0 + 7) // 8

            def step(i, acc, s0=s0, s1=s1):
                base = s0 + i * 8
                rows = h_ref[pl.ds(base, 8), :]
                mask = (base + lax.broadcasted_iota(
                    jnp.int32, (8, 1), 0)) < s1
                return jnp.maximum(acc, jnp.where(mask, rows, -jnp.inf))

            acc = lax.fori_loop(0, nblk, step,
                                jnp.full((8, h), -jnp.inf, jnp.float32))
            o_ref[r, :] = jnp.max(acc, axis=0)

    grid_spec = pltpu.PrefetchScalarGridSpec(
        num_scalar_prefetch=1,
        grid=(g // gpb,),
        in_specs=[pl.BlockSpec((nrp, h), lambda i, st: (0, 0))],
        out_specs=pl.BlockSpec((gpb, h), lambda i, st: (i, 0)),
    )
    return pl.pallas_call(
        body, grid_spec=grid_spec,
        out_shape=jax.ShapeDtypeStruct((g, h), jnp.float32))(starts, h2p)


def _tc_final(pooled, wlin, blin):
    g = pooled.shape[0]
    cc = wlin.shape[1]

    def body(p_ref, w_ref, b_ref, o_ref):
        o_ref[...] = jnp.dot(p_ref[...], w_ref[...],
                             preferred_element_type=jnp.float32) + b_ref[...]

    return pl.pallas_call(
        body, out_shape=jax.ShapeDtypeStruct((g, cc), jnp.float32))(
            pooled, wlin, blin)


def kernel(x, edge_index, batch, W1, b1, W2, b2, Wlin, blin):
    n, d = x.shape
    h = W1.shape[1]
    e = edge_index.shape[1]
    g = NGRAPHS

    nra = NS * 128
    nr = ((n + 8 + nra - 1) // nra) * nra

    e_per_w = -(-e // NW)
    k_ch = -(-e_per_w // CHUNK)
    k_ch = max(2 * NBUF, -(-k_ch // NBUF) * NBUF)
    e_pad = NW * k_ch * CHUNK
    npad = e_pad - e
    pidx = jnp.arange(npad, dtype=jnp.int32)
    src = jnp.concatenate([edge_index[0], pidx % jnp.int32(n)])
    dst = jnp.concatenate([edge_index[1], jnp.int32(n) + pidx % jnp.int32(nr - n)])
    src3 = src.reshape(NW, k_ch, CHUNK)
    dst3 = dst.reshape(NW, k_ch, CHUNK)
    nreal = e // CHUNK
    pad_dst = jnp.concatenate([
        edge_index[1, nreal * CHUNK:],
        jnp.int32(n) + pidx % jnp.int32(nr - n)])
    xp = jnp.concatenate([x, jnp.zeros((nr - n, d), jnp.float32)])
    b1r = b1.reshape(1, h)
    b2r = b2.reshape(1, h)
    blinr = blin.reshape(1, -1)
    starts = jnp.searchsorted(
        batch, jnp.arange(g + 1, dtype=jnp.int32), side="left"
    ).astype(jnp.int32)

    deg = _deg_call(nr, k_ch, nreal)(edge_index, pad_dst)
    deg2 = deg.reshape(NC * nr, 1)
    xw1 = _tc_matmul(xp, W1)
    y1 = _tc_scale(deg2, xw1)
    agg1 = _agg_call(nr, k_ch, h)(y1, src3, dst3)
    y2 = _tc_layer_mid(agg1, y1, deg2, W2, b1r)
    agg2 = _agg_call(nr, k_ch, h)(y2, src3, dst3)
    h2 = _tc_layer_last(agg2, y2, deg2, b2r)
    pooled = _tc_pool(h2, starts, g)
    return _tc_final(pooled, Wlin, blinr)
